# Initial kernel scaffold; baseline (speedup 1.0000x reference)
import jax
import jax.numpy as jnp
from jax.experimental import pallas as pl


def kernel(pred, anchors, gt_boxes, gt_classes):
    raise NotImplementedError("write your pallas kernel here")


# speedup vs baseline: 20.3306x; 20.3306x over previous
import functools

import jax
import jax.numpy as jnp
from jax import lax
from jax.experimental import pallas as pl

B = 16
G = 50
H = 52
W = 52
HW = H * W
A = 3
C = 80
IOU_THRESHOLD = 0.7
LAMBDA_OBJ = 5.0
LAMBDA_NOOBJ = 1.0
LAMBDA_COORD = 1.0


def _cell_prep(cx, cy, w, h, awv):
    fx = jnp.clip(jnp.floor(cx * W), 0.0, W - 1)
    fy = jnp.clip(jnp.floor(cy * H), 0.0, H - 1)
    cxi = fx.astype(jnp.int32)
    cyi = fy.astype(jnp.int32)
    cellidx = cyi * W + cxi
    best = None
    bp = None
    for a in range(A):
        aww = awv[a : a + 1, 0:1]
        awh = awv[a : a + 1, 1:2]
        inter = jnp.minimum(w, aww) * jnp.minimum(h, awh)
        un = w * h + aww * awh - inter
        iou = inter / jnp.clip(un, 1e-9, None)
        if a == 0:
            best = iou
            bp = jnp.zeros_like(cellidx)
        else:
            bp = jnp.where(iou > best, a, bp)
            best = jnp.maximum(iou, best)
    return fx, fy, cxi, cyi, cellidx, bp


def _loss_kernel(pr_ref, gtb_ref, gtbT_ref, clsr_ref, aw_ref, out_ref):
    b = pl.program_id(0)
    awv = aw_ref[...]

    cx = gtb_ref[0, :, 0:1]
    cy = gtb_ref[0, :, 1:2]
    w = jnp.clip(gtb_ref[0, :, 2:3], 1e-4, None)
    h = jnp.clip(gtb_ref[0, :, 3:4], 1e-4, None)
    _, _, _, _, cellidx, bp = _cell_prep(cx, cy, w, h, awv)
    gx1 = cx - w * 0.5
    gx2 = cx + w * 0.5
    gy1 = cy - h * 0.5
    gy2 = cy + h * 0.5
    area_g = w * h

    cx_r = gtbT_ref[0, 0:1, :]
    cy_r = gtbT_ref[0, 1:2, :]
    w_r = jnp.clip(gtbT_ref[0, 2:3, :], 1e-4, None)
    h_r = jnp.clip(gtbT_ref[0, 3:4, :], 1e-4, None)
    fx_r, fy_r, _, _, cellidx_r, bp_r = _cell_prep(cx_r, cy_r, w_r, h_r, awv)
    dx_r = cx_r - fx_r * (1.0 / W)
    dy_r = cy_r - fy_r * (1.0 / H)
    awselw = jnp.where(bp_r == 0, awv[0:1, 0:1],
                       jnp.where(bp_r == 1, awv[1:2, 0:1], awv[2:3, 0:1]))
    awselh = jnp.where(bp_r == 0, awv[0:1, 1:2],
                       jnp.where(bp_r == 1, awv[1:2, 1:2], awv[2:3, 1:2]))
    gw_r = jnp.log(w_r) - jnp.log(awselw)
    gh_r = jnp.log(h_r) - jnp.log(awselh)
    clsid_r = clsr_ref[0].astype(jnp.int32)

    iota_n = lax.broadcasted_iota(jnp.int32, (1, HW), 1)
    gxv = (iota_n % W).astype(jnp.float32) * (1.0 / W)
    gyv = (iota_n // W).astype(jnp.float32) * (1.0 / H)
    iota_n2 = lax.broadcasted_iota(jnp.int32, (HW, 1), 0)
    iota_cls = lax.broadcasted_iota(jnp.int32, (C, 1), 0)

    noobj = jnp.float32(0.0)
    tgt_iou = jnp.zeros((G, 1), jnp.float32)
    obg = jnp.zeros((G, 1), jnp.float32)
    gfeat = jnp.zeros((4, G), jnp.float32)
    gcls = jnp.zeros((C, G), jnp.float32)

    for a in range(A):
        blk = pr_ref[0, pl.ds(85 * a, 5), :]
        xs = jax.nn.sigmoid(blk[0:1])
        ys = jax.nn.sigmoid(blk[1:2])
        tw = blk[2:3]
        th = blk[3:4]
        ob = jax.nn.sigmoid(blk[4:5])
        pwv = awv[a : a + 1, 0:1] * jnp.exp(tw)
        phv = awv[a : a + 1, 1:2] * jnp.exp(th)
        pcx = xs + gxv
        pcy = ys + gyv
        px1 = pcx - pwv * 0.5
        px2 = pcx + pwv * 0.5
        py1 = pcy - phv * 0.5
        py2 = pcy + phv * 0.5

        iw = jnp.clip(jnp.minimum(gx2, px2) - jnp.maximum(gx1, px1), 0.0, None)
        ih = jnp.clip(jnp.minimum(gy2, py2) - jnp.maximum(gy1, py1), 0.0, None)
        inter = iw * ih
        un = area_g + pwv * phv - inter
        iou = inter / jnp.clip(un, 1e-9, None)
        maxiou = jnp.max(iou, axis=0, keepdims=True)

        Mb = (iota_n == cellidx) & (bp == a)
        Mf = Mb.astype(jnp.float32)
        tgt_iou = tgt_iou + jnp.sum(iou * Mf, axis=1, keepdims=True)
        obg = obg + jnp.sum(ob * Mf, axis=1, keepdims=True)
        assigned = jnp.sum(Mf, axis=0, keepdims=True)
        negmask = (maxiou <= IOU_THRESHOLD) & (assigned <= 0.0)
        noobj = noobj + jnp.sum(jnp.where(negmask, ob * ob, 0.0))

        Mt = ((iota_n2 == cellidx_r) & (bp_r == a)).astype(jnp.float32)
        feats = jnp.concatenate([xs, ys, tw, th], axis=0)
        gfeat = gfeat + lax.dot_general(
            feats, Mt, (((1,), (0,)), ((), ())),
            preferred_element_type=jnp.float32,
            precision=lax.Precision.HIGHEST)
        clsblk = pr_ref[0, pl.ds(85 * a + 5, C), :]
        gcls = gcls + lax.dot_general(
            clsblk, Mt, (((1,), (0,)), ((), ())),
            preferred_element_type=jnp.float32,
            precision=lax.Precision.HIGHEST)

    sx = gfeat[0:1]
    sy = gfeat[1:2]
    gtw = gfeat[2:3]
    gth = gfeat[3:4]
    coord = jnp.sum((sx - dx_r) ** 2 + (sy - dy_r) ** 2
                    + (gtw - gw_r) ** 2 + (gth - gh_r) ** 2)
    obj = jnp.sum((obg - tgt_iou) ** 2)
    oh = (iota_cls == clsid_r).astype(jnp.float32)
    scls = jax.nn.sigmoid(gcls)
    cls_loss = jnp.sum((scls - oh) ** 2)

    total = (cls_loss + LAMBDA_NOOBJ * noobj + LAMBDA_OBJ * obj
             + LAMBDA_COORD * coord)
    prev = jnp.where(b == 0, jnp.zeros((1, 1), jnp.float32), out_ref[...])
    out_ref[...] = prev + total


@jax.jit
def kernel(pred, anchors, gt_boxes, gt_classes):
    pr = pred.reshape(B, A * 85, HW)
    gtb = gt_boxes
    gtbT = gt_boxes.transpose(0, 2, 1)
    clsr = gt_classes.astype(jnp.float32).reshape(B, 1, G)
    aw2 = anchors.reshape(A, 2)

    out = pl.pallas_call(
        _loss_kernel,
        grid=(B,),
        in_specs=[
            pl.BlockSpec((1, A * 85, HW), lambda b: (b, 0, 0)),
            pl.BlockSpec((1, G, 4), lambda b: (b, 0, 0)),
            pl.BlockSpec((1, 4, G), lambda b: (b, 0, 0)),
            pl.BlockSpec((1, 1, G), lambda b: (b, 0, 0)),
            pl.BlockSpec((A, 2), lambda b: (0, 0)),
        ],
        out_specs=pl.BlockSpec((1, 1), lambda b: (0, 0)),
        out_shape=jax.ShapeDtypeStruct((1, 1), jnp.float32),
    )(pr, gtb, gtbT, clsr, aw2)
    return out[0, 0]
